# baseline (speedup 1.0000x reference)
import functools

import jax
import jax.numpy as jnp
from jax import lax
from jax.experimental import pallas as pl
from jax.experimental.pallas import tpu as pltpu
from jax.experimental.pallas import tpu_sc as plsc

N = 100000
E = 6400000
NCLS = 10
D = 16
ALPHA = 0.9

NC = 2
NS = 16
NW = NC * NS

BLK = 2048
SUB = BLK // 128
NBLK = E // BLK
ITERS = (NBLK + NW - 1) // NW

EBLK = 640
ESUB = EBLK // 128
ENBLK = E // EBLK
EITERS = (ENBLK + NW - 1) // NW
EPAIRS = (EITERS + 2 + 1) // 2

RPS = 6272
NP = NS * RPS
TAIL = RPS - 3 * BLK
EZC = RPS // EBLK
ETAIL = RPS - EZC * EBLK

_mesh = plsc.VectorSubcoreMesh(core_axis_name="c", subcore_axis_name="s")
_sc_params = pltpu.CompilerParams(use_tc_tiling_on_sc=False)


@functools.partial(
    pl.kernel,
    out_type=jax.ShapeDtypeStruct((NC, NP), jnp.float32),
    mesh=_mesh,
    scratch_types=[
        pltpu.VMEM((SUB, 128), jnp.int32),
        pltpu.VMEM((BLK,), jnp.float32),
        pltpu.VMEM_SHARED((NP,), jnp.float32),
    ],
    compiler_params=_sc_params,
)
def _deg_pass(col_hbm, deg_hbm, cbuf, ones_v, deg_sp):
    c = lax.axis_index("c")
    s = lax.axis_index("s")
    wid = s * NC + c
    base = s * RPS

    def zrow(i, _):
        ones_v[pl.ds(i * 16, 16)] = jnp.zeros((16,), jnp.float32)
        return 0

    lax.fori_loop(0, BLK // 16, zrow, 0)
    for t in range(3):
        pltpu.sync_copy(ones_v, deg_sp.at[pl.ds(base + t * BLK, BLK)])
    pltpu.sync_copy(ones_v.at[pl.ds(0, TAIL)],
                    deg_sp.at[pl.ds(base + 3 * BLK, TAIL)])

    def orow(i, _):
        ones_v[pl.ds(i * 16, 16)] = jnp.ones((16,), jnp.float32)
        return 0

    lax.fori_loop(0, BLK // 16, orow, 0)
    plsc.subcore_barrier()

    def blk(k, _):
        b = wid + k * NW

        @pl.when(b < NBLK)
        def _():
            pltpu.sync_copy(col_hbm.at[pl.ds(b * SUB, SUB)], cbuf)
            for j in range(SUB):
                pltpu.sync_copy(ones_v.at[pl.ds(j * 128, 128)],
                                deg_sp.at[cbuf.at[j]], add=True)

        return 0

    lax.fori_loop(0, ITERS, blk, 0)
    plsc.subcore_barrier()

    for t in range(3):
        pltpu.sync_copy(deg_sp.at[pl.ds(base + t * BLK, BLK)],
                        deg_hbm.at[c, pl.ds(base + t * BLK, BLK)])
    pltpu.sync_copy(deg_sp.at[pl.ds(base + 3 * BLK, TAIL)],
                    deg_hbm.at[c, pl.ds(base + 3 * BLK, TAIL)])


@functools.partial(
    pl.kernel,
    out_type=jax.ShapeDtypeStruct((NC, NP, D), jnp.float32),
    mesh=_mesh,
    scratch_types=[
        pltpu.VMEM((EBLK,), jnp.int32),
        pltpu.VMEM((2, ESUB, 128), jnp.int32),
        pltpu.VMEM((2, EBLK, D), jnp.float32),
        pltpu.VMEM_SHARED((NP, D), jnp.float32),
        pltpu.SemaphoreType.DMA,
        pltpu.SemaphoreType.DMA,
        pltpu.SemaphoreType.DMA,
    ],
    compiler_params=_sc_params,
)
def _edge_pass(scaled_hbm, row_hbm, col_hbm, acc_hbm, rbuf, cbuf, rows, acc_sp,
               gsem, ssem0, ssem1):
    c = lax.axis_index("c")
    s = lax.axis_index("s")
    wid = s * NC + c
    base = s * RPS
    ssems = (ssem0, ssem1)

    def zrow(i, _):
        rows[0, i, :] = jnp.zeros((16,), jnp.float32)
        return 0

    lax.fori_loop(0, EBLK, zrow, 0)
    for t in range(EZC):
        pltpu.sync_copy(rows.at[0], acc_sp.at[pl.ds(base + t * EBLK, EBLK)])
    pltpu.sync_copy(rows.at[0, pl.ds(0, ETAIL)],
                    acc_sp.at[pl.ds(base + EZC * EBLK, ETAIL)])
    plsc.subcore_barrier()

    def one_block(k, p):
        b = wid + k * NW
        bd = b - 2 * NW

        del bd
        @pl.when(b < ENBLK)
        def _():
            pltpu.sync_copy(row_hbm.at[pl.ds(b * EBLK, EBLK)], rbuf)
            pltpu.sync_copy(col_hbm.at[pl.ds(b * ESUB, ESUB)], cbuf.at[p])
            cps = [
                pltpu.async_copy(scaled_hbm.at[rbuf.at[pl.ds(j * 128, 128)]],
                                 rows.at[p, pl.ds(j * 128, 128)], gsem)
                for j in range(ESUB)
            ]
            for cp in cps:
                cp.wait()
            scps = [
                pltpu.async_copy(rows.at[p, pl.ds(j * 128, 128)],
                                 acc_sp.at[cbuf.at[p, j]], ssems[p], add=True)
                for j in range(ESUB)
            ]
            for cp in scps:
                cp.wait()

    def pair(k2, _):
        one_block(2 * k2, 0)
        one_block(2 * k2 + 1, 1)
        return 0

    lax.fori_loop(0, EPAIRS, pair, 0)
    plsc.subcore_barrier()

    for t in range(EZC):
        pltpu.sync_copy(acc_sp.at[pl.ds(base + t * EBLK, EBLK)],
                        acc_hbm.at[c, pl.ds(base + t * EBLK, EBLK)])
    pltpu.sync_copy(acc_sp.at[pl.ds(base + EZC * EBLK, ETAIL)],
                    acc_hbm.at[c, pl.ds(base + EZC * EBLK, ETAIL)])


BT = 2000
_GRID = N // BT


def _init_body(y_ref, m_ref, d0_ref, d1_ref, seed_ref, dinv_ref, scaled_ref):
    cls = lax.broadcasted_iota(jnp.int32, (BT, D), 1)
    seed = jnp.where((cls == y_ref[...]) & (m_ref[...] > 0.0), 1.0, 0.0)
    seed = seed.astype(jnp.float32)
    deg = d0_ref[...] + d1_ref[...]
    dinv = jnp.where(deg > 0.0, lax.rsqrt(jnp.maximum(deg, 1e-12)), 0.0)
    dinv = jnp.broadcast_to(dinv, (BT, D))
    seed_ref[...] = seed
    dinv_ref[...] = dinv
    scaled_ref[...] = dinv * seed


_col_spec = pl.BlockSpec((BT, 1), lambda i: (i, 0))
_tab_spec = pl.BlockSpec((BT, D), lambda i: (i, 0))

_init_call = pl.pallas_call(
    _init_body,
    grid=(_GRID,),
    in_specs=[_col_spec, _col_spec, _col_spec, _col_spec],
    out_specs=[_tab_spec, _tab_spec, _tab_spec],
    out_shape=[jax.ShapeDtypeStruct((N, D), jnp.float32)] * 3,
)


def _comb_body(a0_ref, a1_ref, dv_ref, seed_ref, out_ref, scaled_ref):
    acc = a0_ref[...] + a1_ref[...]
    dv = dv_ref[...]
    o = ALPHA * (dv * acc) + (1.0 - ALPHA) * seed_ref[...]
    o = jnp.clip(o, 0.0, 1.0)
    out_ref[...] = o
    scaled_ref[...] = dv * o


_comb_call = pl.pallas_call(
    _comb_body,
    grid=(_GRID,),
    in_specs=[_tab_spec, _tab_spec, _tab_spec, _tab_spec],
    out_specs=[_tab_spec, _tab_spec],
    out_shape=[jax.ShapeDtypeStruct((N, D), jnp.float32)] * 2,
)


def kernel(y, edge_index, train_mask, edge_weight):
    del edge_weight
    row = edge_index[0]
    col2d = edge_index[1].reshape(E // 128, 128)

    deg2 = _deg_pass(col2d)
    d0 = deg2[0, :N].reshape(N, 1)
    d1 = deg2[1, :N].reshape(N, 1)
    y1 = y.reshape(N, 1).astype(jnp.int32)
    m1 = train_mask.reshape(N, 1).astype(jnp.float32)

    seed, dinv, scaled = _init_call(y1, m1, d0, d1)
    out = seed
    for _ in range(3):
        acc = _edge_pass(scaled, row, col2d)
        out, scaled = _comb_call(acc[0, :N], acc[1, :N], dinv, seed)
    return out[:, :NCLS]


# speedup vs baseline: 1.1056x; 1.1056x over previous
import functools

import jax
import jax.numpy as jnp
from jax import lax
from jax.experimental import pallas as pl
from jax.experimental.pallas import tpu as pltpu
from jax.experimental.pallas import tpu_sc as plsc

N = 100000
E = 6400000
NCLS = 10
D = 16
ALPHA = 0.9

NC = 2
NS = 16
NW = NC * NS

BLK = 2048
SUB = BLK // 128
NBLK = E // BLK
ITERS = (NBLK + NW - 1) // NW

EBLK = 640
ESUB = EBLK // 128
ENBLK = E // EBLK
EITERS = (ENBLK + NW - 1) // NW
EPAIRS = (EITERS + 2 + 1) // 2

RPS = 6272
NP = NS * RPS
TAIL = RPS - 3 * BLK
EZC = RPS // EBLK
ETAIL = RPS - EZC * EBLK

_mesh = plsc.VectorSubcoreMesh(core_axis_name="c", subcore_axis_name="s")
_sc_params = pltpu.CompilerParams(use_tc_tiling_on_sc=False)


@functools.partial(
    pl.kernel,
    out_type=jax.ShapeDtypeStruct((NC, NP), jnp.float32),
    mesh=_mesh,
    scratch_types=[
        pltpu.VMEM((SUB, 128), jnp.int32),
        pltpu.VMEM((BLK,), jnp.float32),
        pltpu.VMEM_SHARED((NP,), jnp.float32),
    ],
    compiler_params=_sc_params,
)
def _deg_pass(col_hbm, deg_hbm, cbuf, ones_v, deg_sp):
    c = lax.axis_index("c")
    s = lax.axis_index("s")
    wid = s * NC + c
    base = s * RPS

    def zrow(i, _):
        ones_v[pl.ds(i * 16, 16)] = jnp.zeros((16,), jnp.float32)
        return 0

    lax.fori_loop(0, BLK // 16, zrow, 0)
    for t in range(3):
        pltpu.sync_copy(ones_v, deg_sp.at[pl.ds(base + t * BLK, BLK)])
    pltpu.sync_copy(ones_v.at[pl.ds(0, TAIL)],
                    deg_sp.at[pl.ds(base + 3 * BLK, TAIL)])

    def orow(i, _):
        ones_v[pl.ds(i * 16, 16)] = jnp.ones((16,), jnp.float32)
        return 0

    lax.fori_loop(0, BLK // 16, orow, 0)
    plsc.subcore_barrier()

    def blk(k, _):
        b = wid + k * NW

        @pl.when(b < NBLK)
        def _():
            pltpu.sync_copy(col_hbm.at[pl.ds(b * SUB, SUB)], cbuf)
            for j in range(SUB):
                pltpu.sync_copy(ones_v.at[pl.ds(j * 128, 128)],
                                deg_sp.at[cbuf.at[j]], add=True)

        return 0

    lax.fori_loop(0, ITERS, blk, 0)
    plsc.subcore_barrier()

    for t in range(3):
        pltpu.sync_copy(deg_sp.at[pl.ds(base + t * BLK, BLK)],
                        deg_hbm.at[c, pl.ds(base + t * BLK, BLK)])
    pltpu.sync_copy(deg_sp.at[pl.ds(base + 3 * BLK, TAIL)],
                    deg_hbm.at[c, pl.ds(base + 3 * BLK, TAIL)])


@functools.partial(
    pl.kernel,
    out_type=jax.ShapeDtypeStruct((NC, NP, D), jnp.float32),
    mesh=_mesh,
    scratch_types=[
        pltpu.VMEM((EBLK,), jnp.int32),
        pltpu.VMEM((2, ESUB, 128), jnp.int32),
        pltpu.VMEM((2, EBLK, D), jnp.float32),
        pltpu.VMEM_SHARED((NP, D), jnp.float32),
        pltpu.SemaphoreType.DMA,
        pltpu.SemaphoreType.DMA,
        pltpu.SemaphoreType.DMA,
    ],
    compiler_params=_sc_params,
)
def _edge_pass(scaled_hbm, row_hbm, col_hbm, acc_hbm, rbuf, cbuf, rows, acc_sp,
               gsem, ssem0, ssem1):
    c = lax.axis_index("c")
    s = lax.axis_index("s")
    wid = s * NC + c
    base = s * RPS
    ssems = (ssem0, ssem1)

    def zrow(i, _):
        rows[0, i, :] = jnp.zeros((16,), jnp.float32)
        return 0

    lax.fori_loop(0, EBLK, zrow, 0)
    for t in range(EZC):
        pltpu.sync_copy(rows.at[0], acc_sp.at[pl.ds(base + t * EBLK, EBLK)])
    pltpu.sync_copy(rows.at[0, pl.ds(0, ETAIL)],
                    acc_sp.at[pl.ds(base + EZC * EBLK, ETAIL)])
    plsc.subcore_barrier()

    def one_block(k, p):
        b = wid + k * NW
        bd = b - 2 * NW

        @pl.when((k >= 2) & (bd < ENBLK))
        def _():
            for j in range(ESUB):
                pltpu.make_async_copy(rows.at[p, pl.ds(j * 128, 128)],
                                      acc_sp.at[cbuf.at[p, j]],
                                      ssems[p]).wait()

        @pl.when(b < ENBLK)
        def _():
            pltpu.sync_copy(row_hbm.at[pl.ds(b * EBLK, EBLK)], rbuf)
            pltpu.sync_copy(col_hbm.at[pl.ds(b * ESUB, ESUB)], cbuf.at[p])
            cps = [
                pltpu.async_copy(scaled_hbm.at[rbuf.at[pl.ds(j * 128, 128)]],
                                 rows.at[p, pl.ds(j * 128, 128)], gsem)
                for j in range(ESUB)
            ]
            for cp in cps:
                cp.wait()
            for j in range(ESUB):
                pltpu.async_copy(rows.at[p, pl.ds(j * 128, 128)],
                                 acc_sp.at[cbuf.at[p, j]], ssems[p], add=True)

    def pair(k2, _):
        one_block(2 * k2, 0)
        one_block(2 * k2 + 1, 1)
        return 0

    lax.fori_loop(0, EPAIRS, pair, 0)
    plsc.subcore_barrier()

    for t in range(EZC):
        pltpu.sync_copy(acc_sp.at[pl.ds(base + t * EBLK, EBLK)],
                        acc_hbm.at[c, pl.ds(base + t * EBLK, EBLK)])
    pltpu.sync_copy(acc_sp.at[pl.ds(base + EZC * EBLK, ETAIL)],
                    acc_hbm.at[c, pl.ds(base + EZC * EBLK, ETAIL)])


BT = 2000
_GRID = N // BT


def _init_body(y_ref, m_ref, d0_ref, d1_ref, seed_ref, dinv_ref, scaled_ref):
    cls = lax.broadcasted_iota(jnp.int32, (BT, D), 1)
    seed = jnp.where((cls == y_ref[...]) & (m_ref[...] > 0.0), 1.0, 0.0)
    seed = seed.astype(jnp.float32)
    deg = d0_ref[...] + d1_ref[...]
    dinv = jnp.where(deg > 0.0, lax.rsqrt(jnp.maximum(deg, 1e-12)), 0.0)
    dinv = jnp.broadcast_to(dinv, (BT, D))
    seed_ref[...] = seed
    dinv_ref[...] = dinv
    scaled_ref[...] = dinv * seed


_col_spec = pl.BlockSpec((BT, 1), lambda i: (i, 0))
_tab_spec = pl.BlockSpec((BT, D), lambda i: (i, 0))

_init_call = pl.pallas_call(
    _init_body,
    grid=(_GRID,),
    in_specs=[_col_spec, _col_spec, _col_spec, _col_spec],
    out_specs=[_tab_spec, _tab_spec, _tab_spec],
    out_shape=[jax.ShapeDtypeStruct((N, D), jnp.float32)] * 3,
)


def _comb_body(a0_ref, a1_ref, dv_ref, seed_ref, out_ref, scaled_ref):
    acc = a0_ref[...] + a1_ref[...]
    dv = dv_ref[...]
    o = ALPHA * (dv * acc) + (1.0 - ALPHA) * seed_ref[...]
    o = jnp.clip(o, 0.0, 1.0)
    out_ref[...] = o
    scaled_ref[...] = dv * o


_comb_call = pl.pallas_call(
    _comb_body,
    grid=(_GRID,),
    in_specs=[_tab_spec, _tab_spec, _tab_spec, _tab_spec],
    out_specs=[_tab_spec, _tab_spec],
    out_shape=[jax.ShapeDtypeStruct((N, D), jnp.float32)] * 2,
)


def kernel(y, edge_index, train_mask, edge_weight):
    del edge_weight
    row = edge_index[0]
    col2d = edge_index[1].reshape(E // 128, 128)

    deg2 = _deg_pass(col2d)
    d0 = deg2[0, :N].reshape(N, 1)
    d1 = deg2[1, :N].reshape(N, 1)
    y1 = y.reshape(N, 1).astype(jnp.int32)
    m1 = train_mask.reshape(N, 1).astype(jnp.float32)

    seed, dinv, scaled = _init_call(y1, m1, d0, d1)
    out = seed
    for _ in range(3):
        acc = _edge_pass(scaled, row, col2d)
        out, scaled = _comb_call(acc[0, :N], acc[1, :N], dinv, seed)
    return out[:, :NCLS]


# speedup vs baseline: 1.4628x; 1.3231x over previous
import functools

import jax
import jax.numpy as jnp
from jax import lax
from jax.experimental import pallas as pl
from jax.experimental.pallas import tpu as pltpu
from jax.experimental.pallas import tpu_sc as plsc

N = 100000
E = 6400000
NCLS = 10
D = 16
ALPHA = 0.9

NC = 2
NS = 16
NW = NC * NS

BLK = 2048
SUB = BLK // 128
NBLK = E // BLK
ITERS = (NBLK + NW - 1) // NW

EBLK = 512
ESUB = EBLK // 128
ENBLK = E // EBLK
EITERS = (ENBLK + NW - 1) // NW
ETRIPS = (EITERS + 2 + 2) // 3

RPS = 6272
NP = NS * RPS
TAIL = RPS - 3 * BLK
EZC = RPS // EBLK
ETAIL = RPS - EZC * EBLK

_mesh = plsc.VectorSubcoreMesh(core_axis_name="c", subcore_axis_name="s")
_sc_params = pltpu.CompilerParams(use_tc_tiling_on_sc=False)


@functools.partial(
    pl.kernel,
    out_type=jax.ShapeDtypeStruct((NC, NP), jnp.float32),
    mesh=_mesh,
    scratch_types=[
        pltpu.VMEM((SUB, 128), jnp.int32),
        pltpu.VMEM((BLK,), jnp.float32),
        pltpu.VMEM_SHARED((NP,), jnp.float32),
    ],
    compiler_params=_sc_params,
)
def _deg_pass(col_hbm, deg_hbm, cbuf, ones_v, deg_sp):
    c = lax.axis_index("c")
    s = lax.axis_index("s")
    wid = s * NC + c
    base = s * RPS

    def zrow(i, _):
        ones_v[pl.ds(i * 16, 16)] = jnp.zeros((16,), jnp.float32)
        return 0

    lax.fori_loop(0, BLK // 16, zrow, 0)
    for t in range(3):
        pltpu.sync_copy(ones_v, deg_sp.at[pl.ds(base + t * BLK, BLK)])
    pltpu.sync_copy(ones_v.at[pl.ds(0, TAIL)],
                    deg_sp.at[pl.ds(base + 3 * BLK, TAIL)])

    def orow(i, _):
        ones_v[pl.ds(i * 16, 16)] = jnp.ones((16,), jnp.float32)
        return 0

    lax.fori_loop(0, BLK // 16, orow, 0)
    plsc.subcore_barrier()

    def blk(k, _):
        b = wid + k * NW

        @pl.when(b < NBLK)
        def _():
            pltpu.sync_copy(col_hbm.at[pl.ds(b * SUB, SUB)], cbuf)
            for j in range(SUB):
                pltpu.sync_copy(ones_v.at[pl.ds(j * 128, 128)],
                                deg_sp.at[cbuf.at[j]], add=True)

        return 0

    lax.fori_loop(0, ITERS, blk, 0)
    plsc.subcore_barrier()

    for t in range(3):
        pltpu.sync_copy(deg_sp.at[pl.ds(base + t * BLK, BLK)],
                        deg_hbm.at[c, pl.ds(base + t * BLK, BLK)])
    pltpu.sync_copy(deg_sp.at[pl.ds(base + 3 * BLK, TAIL)],
                    deg_hbm.at[c, pl.ds(base + 3 * BLK, TAIL)])


@functools.partial(
    pl.kernel,
    out_type=jax.ShapeDtypeStruct((NC, NP, D), jnp.float32),
    mesh=_mesh,
    scratch_types=[
        pltpu.VMEM((3, EBLK), jnp.int32),
        pltpu.VMEM((3, ESUB, 128), jnp.int32),
        pltpu.VMEM((3, EBLK, D), jnp.float32),
        pltpu.VMEM_SHARED((NP, D), jnp.float32),
        pltpu.SemaphoreType.DMA,
        pltpu.SemaphoreType.DMA,
        pltpu.SemaphoreType.DMA,
        pltpu.SemaphoreType.DMA,
        pltpu.SemaphoreType.DMA,
        pltpu.SemaphoreType.DMA,
    ],
    compiler_params=_sc_params,
)
def _edge_pass(scaled_hbm, row_hbm, col_hbm, acc_hbm, rbuf, cbuf, rows, acc_sp,
               gsem0, gsem1, gsem2, ssem0, ssem1, ssem2):
    c = lax.axis_index("c")
    s = lax.axis_index("s")
    wid = s * NC + c
    base = s * RPS
    gsems = (gsem0, gsem1, gsem2)
    ssems = (ssem0, ssem1, ssem2)

    def zrow(i, _):
        rows[0, i, :] = jnp.zeros((16,), jnp.float32)
        return 0

    lax.fori_loop(0, EBLK, zrow, 0)
    for t in range(EZC):
        pltpu.sync_copy(rows.at[0], acc_sp.at[pl.ds(base + t * EBLK, EBLK)])
    pltpu.sync_copy(rows.at[0, pl.ds(0, ETAIL)],
                    acc_sp.at[pl.ds(base + EZC * EBLK, ETAIL)])
    plsc.subcore_barrier()

    def load_and_fire(b, p):
        pltpu.sync_copy(row_hbm.at[pl.ds(b * EBLK, EBLK)], rbuf.at[p])
        pltpu.sync_copy(col_hbm.at[pl.ds(b * ESUB, ESUB)], cbuf.at[p])
        for j in range(ESUB):
            pltpu.async_copy(scaled_hbm.at[rbuf.at[p, pl.ds(j * 128, 128)]],
                             rows.at[p, pl.ds(j * 128, 128)], gsems[p])

    def one_block(k, p):
        pn = (p + 1) % 3
        b = wid + k * NW
        bd = b - 2 * NW

        @pl.when((k >= 2) & (bd < ENBLK))
        def _():
            for j in range(ESUB):
                pltpu.make_async_copy(rows.at[pn, pl.ds(j * 128, 128)],
                                      acc_sp.at[cbuf.at[pn, j]],
                                      ssems[pn]).wait()

        @pl.when(b + NW < ENBLK)
        def _():
            load_and_fire(b + NW, pn)

        @pl.when(b < ENBLK)
        def _():
            for j in range(ESUB):
                pltpu.make_async_copy(
                    scaled_hbm.at[rbuf.at[p, pl.ds(j * 128, 128)]],
                    rows.at[p, pl.ds(j * 128, 128)], gsems[p]).wait()
            for j in range(ESUB):
                pltpu.async_copy(rows.at[p, pl.ds(j * 128, 128)],
                                 acc_sp.at[cbuf.at[p, j]], ssems[p], add=True)

    load_and_fire(wid, 0)

    def trip(k3, _):
        one_block(3 * k3, 0)
        one_block(3 * k3 + 1, 1)
        one_block(3 * k3 + 2, 2)
        return 0

    lax.fori_loop(0, ETRIPS, trip, 0)
    plsc.subcore_barrier()

    for t in range(EZC):
        pltpu.sync_copy(acc_sp.at[pl.ds(base + t * EBLK, EBLK)],
                        acc_hbm.at[c, pl.ds(base + t * EBLK, EBLK)])
    pltpu.sync_copy(acc_sp.at[pl.ds(base + EZC * EBLK, ETAIL)],
                    acc_hbm.at[c, pl.ds(base + EZC * EBLK, ETAIL)])


BT = 2000
_GRID = N // BT


def _init_body(y_ref, m_ref, d0_ref, d1_ref, seed_ref, dinv_ref, scaled_ref):
    cls = lax.broadcasted_iota(jnp.int32, (BT, D), 1)
    seed = jnp.where((cls == y_ref[...]) & (m_ref[...] > 0.0), 1.0, 0.0)
    seed = seed.astype(jnp.float32)
    deg = d0_ref[...] + d1_ref[...]
    dinv = jnp.where(deg > 0.0, lax.rsqrt(jnp.maximum(deg, 1e-12)), 0.0)
    dinv = jnp.broadcast_to(dinv, (BT, D))
    seed_ref[...] = seed
    dinv_ref[...] = dinv
    scaled_ref[...] = dinv * seed


_col_spec = pl.BlockSpec((BT, 1), lambda i: (i, 0))
_tab_spec = pl.BlockSpec((BT, D), lambda i: (i, 0))

_init_call = pl.pallas_call(
    _init_body,
    grid=(_GRID,),
    in_specs=[_col_spec, _col_spec, _col_spec, _col_spec],
    out_specs=[_tab_spec, _tab_spec, _tab_spec],
    out_shape=[jax.ShapeDtypeStruct((N, D), jnp.float32)] * 3,
)


def _comb_body(a0_ref, a1_ref, dv_ref, seed_ref, out_ref, scaled_ref):
    acc = a0_ref[...] + a1_ref[...]
    dv = dv_ref[...]
    o = ALPHA * (dv * acc) + (1.0 - ALPHA) * seed_ref[...]
    o = jnp.clip(o, 0.0, 1.0)
    out_ref[...] = o
    scaled_ref[...] = dv * o


_comb_call = pl.pallas_call(
    _comb_body,
    grid=(_GRID,),
    in_specs=[_tab_spec, _tab_spec, _tab_spec, _tab_spec],
    out_specs=[_tab_spec, _tab_spec],
    out_shape=[jax.ShapeDtypeStruct((N, D), jnp.float32)] * 2,
)


def kernel(y, edge_index, train_mask, edge_weight):
    del edge_weight
    row = edge_index[0]
    col2d = edge_index[1].reshape(E // 128, 128)

    deg2 = _deg_pass(col2d)
    d0 = deg2[0, :N].reshape(N, 1)
    d1 = deg2[1, :N].reshape(N, 1)
    y1 = y.reshape(N, 1).astype(jnp.int32)
    m1 = train_mask.reshape(N, 1).astype(jnp.float32)

    seed, dinv, scaled = _init_call(y1, m1, d0, d1)
    out = seed
    for _ in range(3):
        acc = _edge_pass(scaled, row, col2d)
        out, scaled = _comb_call(acc[0, :N], acc[1, :N], dinv, seed)
    return out[:, :NCLS]


# speedup vs baseline: 1.6546x; 1.1311x over previous
import functools

import jax
import jax.numpy as jnp
from jax import lax
from jax.experimental import pallas as pl
from jax.experimental.pallas import tpu as pltpu
from jax.experimental.pallas import tpu_sc as plsc

N = 100000
E = 6400000
NCLS = 10
D = 16
ALPHA = 0.9

NC = 2
NS = 16
NW = NC * NS

BLK = 2048
SUB = BLK // 128
NBLK = E // BLK
ITERS = (NBLK + NW - 1) // NW

EBLK = 512
ESUB = EBLK // 128
ENBLK = E // EBLK
EITERS = (ENBLK + NW - 1) // NW
ETRIPS = (EITERS + 2 + 2) // 3

RPS = 6272
NP = NS * RPS
TAIL = RPS - 3 * BLK
EZC = RPS // EBLK
ETAIL = RPS - EZC * EBLK

_mesh = plsc.VectorSubcoreMesh(core_axis_name="c", subcore_axis_name="s")
_sc_params = pltpu.CompilerParams(use_tc_tiling_on_sc=False)


@functools.partial(
    pl.kernel,
    out_type=jax.ShapeDtypeStruct((NC, NP), jnp.float32),
    mesh=_mesh,
    scratch_types=[
        pltpu.VMEM((SUB, 128), jnp.int32),
        pltpu.VMEM((BLK,), jnp.float32),
        pltpu.VMEM_SHARED((NP,), jnp.float32),
        pltpu.SemaphoreType.DMA,
    ],
    compiler_params=_sc_params,
)
def _deg_pass(col_hbm, deg_hbm, cbuf, ones_v, deg_sp, dsem):
    c = lax.axis_index("c")
    s = lax.axis_index("s")
    wid = s * NC + c
    base = s * RPS

    def zrow(i, _):
        ones_v[pl.ds(i * 16, 16)] = jnp.zeros((16,), jnp.float32)
        return 0

    lax.fori_loop(0, BLK // 16, zrow, 0)
    for t in range(3):
        pltpu.sync_copy(ones_v, deg_sp.at[pl.ds(base + t * BLK, BLK)])
    pltpu.sync_copy(ones_v.at[pl.ds(0, TAIL)],
                    deg_sp.at[pl.ds(base + 3 * BLK, TAIL)])

    def orow(i, _):
        ones_v[pl.ds(i * 16, 16)] = jnp.ones((16,), jnp.float32)
        return 0

    lax.fori_loop(0, BLK // 16, orow, 0)
    plsc.subcore_barrier()

    def blk(k, _):
        b = wid + k * NW

        @pl.when(b < NBLK)
        def _():
            pltpu.sync_copy(col_hbm.at[pl.ds(b * SUB, SUB)], cbuf)
            scps = [
                pltpu.async_copy(ones_v.at[pl.ds(j * 128, 128)],
                                 deg_sp.at[cbuf.at[j]], dsem, add=True)
                for j in range(SUB)
            ]
            for cp in scps:
                cp.wait()

        return 0

    lax.fori_loop(0, ITERS, blk, 0)
    plsc.subcore_barrier()

    for t in range(3):
        pltpu.sync_copy(deg_sp.at[pl.ds(base + t * BLK, BLK)],
                        deg_hbm.at[c, pl.ds(base + t * BLK, BLK)])
    pltpu.sync_copy(deg_sp.at[pl.ds(base + 3 * BLK, TAIL)],
                    deg_hbm.at[c, pl.ds(base + 3 * BLK, TAIL)])


@functools.partial(
    pl.kernel,
    out_type=jax.ShapeDtypeStruct((NC, NP, D), jnp.float32),
    mesh=_mesh,
    scratch_types=[
        pltpu.VMEM((3, EBLK), jnp.int32),
        pltpu.VMEM((3, ESUB, 128), jnp.int32),
        pltpu.VMEM((3, EBLK, D), jnp.float32),
        pltpu.VMEM_SHARED((NP, D), jnp.float32),
        pltpu.SemaphoreType.DMA,
        pltpu.SemaphoreType.DMA,
        pltpu.SemaphoreType.DMA,
        pltpu.SemaphoreType.DMA,
        pltpu.SemaphoreType.DMA,
        pltpu.SemaphoreType.DMA,
    ],
    compiler_params=_sc_params,
)
def _edge_pass(scaled_hbm, row_hbm, col_hbm, acc_hbm, rbuf, cbuf, rows, acc_sp,
               gsem0, gsem1, gsem2, ssem0, ssem1, ssem2):
    c = lax.axis_index("c")
    s = lax.axis_index("s")
    wid = s * NC + c
    base = s * RPS
    gsems = (gsem0, gsem1, gsem2)
    ssems = (ssem0, ssem1, ssem2)

    def zrow(i, _):
        rows[0, i, :] = jnp.zeros((16,), jnp.float32)
        return 0

    lax.fori_loop(0, EBLK, zrow, 0)
    for t in range(EZC):
        pltpu.sync_copy(rows.at[0], acc_sp.at[pl.ds(base + t * EBLK, EBLK)])
    pltpu.sync_copy(rows.at[0, pl.ds(0, ETAIL)],
                    acc_sp.at[pl.ds(base + EZC * EBLK, ETAIL)])
    plsc.subcore_barrier()

    def load_and_fire(b, p):
        pltpu.sync_copy(row_hbm.at[pl.ds(b * EBLK, EBLK)], rbuf.at[p])
        pltpu.sync_copy(col_hbm.at[pl.ds(b * ESUB, ESUB)], cbuf.at[p])
        for j in range(ESUB):
            pltpu.async_copy(scaled_hbm.at[rbuf.at[p, pl.ds(j * 128, 128)]],
                             rows.at[p, pl.ds(j * 128, 128)], gsems[p])

    def one_block(k, p):
        pn = (p + 1) % 3
        b = wid + k * NW
        bd = b - 2 * NW

        @pl.when((k >= 2) & (bd < ENBLK))
        def _():
            for j in range(ESUB):
                pltpu.make_async_copy(rows.at[pn, pl.ds(j * 128, 128)],
                                      acc_sp.at[cbuf.at[pn, j]],
                                      ssems[pn]).wait()

        @pl.when(b + NW < ENBLK)
        def _():
            load_and_fire(b + NW, pn)

        @pl.when(b < ENBLK)
        def _():
            for j in range(ESUB):
                pltpu.make_async_copy(
                    scaled_hbm.at[rbuf.at[p, pl.ds(j * 128, 128)]],
                    rows.at[p, pl.ds(j * 128, 128)], gsems[p]).wait()
            for j in range(ESUB):
                pltpu.async_copy(rows.at[p, pl.ds(j * 128, 128)],
                                 acc_sp.at[cbuf.at[p, j]], ssems[p], add=True)

    load_and_fire(wid, 0)

    def trip(k3, _):
        one_block(3 * k3, 0)
        one_block(3 * k3 + 1, 1)
        one_block(3 * k3 + 2, 2)
        return 0

    lax.fori_loop(0, ETRIPS, trip, 0)
    plsc.subcore_barrier()

    for t in range(EZC):
        pltpu.sync_copy(acc_sp.at[pl.ds(base + t * EBLK, EBLK)],
                        acc_hbm.at[c, pl.ds(base + t * EBLK, EBLK)])
    pltpu.sync_copy(acc_sp.at[pl.ds(base + EZC * EBLK, ETAIL)],
                    acc_hbm.at[c, pl.ds(base + EZC * EBLK, ETAIL)])


BT = 2048
assert NP % BT == 0

NPR = NP * D // 128
BTR = 1568
_RGRID = NPR // BTR


def _init_body(y_ref, m_ref, d0_ref, d1_ref, seed_ref, dinv_ref, scaled_ref):
    cls = lax.broadcasted_iota(jnp.int32, (BT, D), 1)
    seed = jnp.where((cls == y_ref[...]) & (m_ref[...] > 0.0), 1.0, 0.0)
    seed = seed.astype(jnp.float32)
    deg = d0_ref[...] + d1_ref[...]
    dinv = jnp.where(deg > 0.0, lax.rsqrt(jnp.maximum(deg, 1e-12)), 0.0)
    dinv = jnp.broadcast_to(dinv, (BT, D))
    seed_ref[...] = seed
    dinv_ref[...] = dinv
    scaled_ref[...] = dinv * seed


_col_spec = pl.BlockSpec((BT, 1), lambda i: (i, 0))
_tab_spec = pl.BlockSpec((BT, D), lambda i: (i, 0))

_init_call = pl.pallas_call(
    _init_body,
    grid=(NP // BT,),
    in_specs=[_col_spec, _col_spec, _col_spec, _col_spec],
    out_specs=[_tab_spec, _tab_spec, _tab_spec],
    out_shape=[jax.ShapeDtypeStruct((NP, D), jnp.float32)] * 3,
)


def _comb_body(a0_ref, a1_ref, dv_ref, seed_ref, out_ref, scaled_ref):
    acc = a0_ref[...] + a1_ref[...]
    dv = dv_ref[...]
    o = ALPHA * (dv * acc) + (1.0 - ALPHA) * seed_ref[...]
    o = jnp.clip(o, 0.0, 1.0)
    out_ref[...] = o
    scaled_ref[...] = dv * o


_r_spec = pl.BlockSpec((BTR, 128), lambda i: (i, 0))

_comb_call = pl.pallas_call(
    _comb_body,
    grid=(_RGRID,),
    in_specs=[_r_spec, _r_spec, _r_spec, _r_spec],
    out_specs=[_r_spec, _r_spec],
    out_shape=[jax.ShapeDtypeStruct((NPR, 128), jnp.float32)] * 2,
)


def kernel(y, edge_index, train_mask, edge_weight):
    del edge_weight
    row = edge_index[0]
    col2d = edge_index[1].reshape(E // 128, 128)

    deg2 = _deg_pass(col2d)
    d0 = deg2[0].reshape(NP, 1)
    d1 = deg2[1].reshape(NP, 1)
    y1 = jnp.pad(y.astype(jnp.int32), (0, NP - N)).reshape(NP, 1)
    m1 = jnp.pad(train_mask.astype(jnp.float32), (0, NP - N)).reshape(NP, 1)

    seed, dinv, scaled = _init_call(y1, m1, d0, d1)
    seed128 = seed.reshape(NPR, 128)
    dinv128 = dinv.reshape(NPR, 128)
    for _ in range(3):
        acc = _edge_pass(scaled, row, col2d)
        acc128 = acc.reshape(NC, NPR, 128)
        out128, scaled128 = _comb_call(acc128[0], acc128[1], dinv128, seed128)
        scaled = scaled128.reshape(NP, D)
    return out128.reshape(NP, D)[:N, :NCLS]
